# baseline (speedup 1.0000x reference)
import jax
import jax.numpy as jnp
from jax import lax
from jax.experimental import pallas as pl
from jax.experimental.pallas import tpu as pltpu
from jax.experimental.pallas import tpu_sc as plsc

NC = 2
NS = 16
SB = 128
K = 12
DP = 8


def _sc_segment_sum(x_pad, rc, zeros, n_acc, total_chunks, c0_chunks):
    zrows = n_acc // NS
    mesh = plsc.VectorSubcoreMesh(
        core_axis_name="c", subcore_axis_name="s",
        num_cores=NC, num_subcores=NS)

    q0, r0 = divmod(c0_chunks, NS)
    q1, r1 = divmod(total_chunks - c0_chunks, NS)
    nslot = max(q0 + (1 if r0 else 0), q1 + (1 if r1 else 0))
    npairs = -(-nslot // 2)

    def body(x_hbm, rc_hbm, zeros_hbm, parts_hbm,
             acc_sh, ridx_v, cidx_v, vals_v,
             si0, si1, sg0, sg1, ss0, ss1):
        c = lax.axis_index("c")
        s = lax.axis_index("s")
        si = (si0, si1)
        sg = (sg0, sg1)
        ss = (ss0, ss1)
        cnt = jnp.where(c == 0, q0 + (s < r0).astype(jnp.int32),
                        q1 + (s < r1).astype(jnp.int32))
        start = jnp.where(c == 0, q0 * s + jnp.minimum(s, r0),
                          c0_chunks + q1 * s + jnp.minimum(s, r1))
        pltpu.sync_copy(zeros_hbm, acc_sh.at[pl.ds(s * zrows, zrows)])
        plsc.subcore_barrier()

        def fire_idx(g, b):
            sb = (start + jnp.minimum(g, cnt - 1)) * K
            pltpu.async_copy(rc_hbm.at[0, pl.ds(sb, K)], ridx_v.at[b], si[b])
            pltpu.async_copy(rc_hbm.at[1, pl.ds(sb, K)], cidx_v.at[b], si[b])

        def wait_idx(b):
            pltpu.make_async_copy(rc_hbm.at[0, pl.ds(0, K)], ridx_v.at[b],
                                  si[b]).wait()
            pltpu.make_async_copy(rc_hbm.at[1, pl.ds(0, K)], cidx_v.at[b],
                                  si[b]).wait()

        def fire_gathers(b):
            for j in range(K):
                pltpu.async_copy(x_hbm.at[ridx_v.at[b, j]], vals_v.at[b, j],
                                 sg[b])

        def wait_gathers(b):
            for j in range(K):
                pltpu.make_async_copy(x_hbm.at[ridx_v.at[b, j]],
                                      vals_v.at[b, j], sg[b]).wait()

        def fire_scatters(b):
            for j in range(K):
                pltpu.async_copy(vals_v.at[b, j], acc_sh.at[cidx_v.at[b, j]],
                                 ss[b], add=True)

        def wait_scatters(b):
            for j in range(K):
                pltpu.make_async_copy(vals_v.at[b, j],
                                      acc_sh.at[cidx_v.at[b, j]],
                                      ss[b]).wait()

        def do_pair(i, first, last):
            g0 = 2 * i
            g1 = g0 + 1
            wait_idx(0)
            if not first:
                pl.when(g0 - 2 < cnt)(lambda: wait_scatters(0))
            pl.when(g0 < cnt)(lambda: fire_gathers(0))
            wait_idx(1)
            if not first:
                pl.when(g1 - 2 < cnt)(lambda: wait_scatters(1))

            def _drain0():
                wait_gathers(0)
                fire_scatters(0)
            pl.when(g0 < cnt)(_drain0)
            pl.when(g1 < cnt)(lambda: fire_gathers(1))
            if not last:
                fire_idx(g0 + 2, 0)

            def _drain1():
                wait_gathers(1)
                fire_scatters(1)
            pl.when(g1 < cnt)(_drain1)
            if not last:
                fire_idx(g1 + 2, 1)

        fire_idx(0, 0)
        fire_idx(1, 1)
        do_pair(0, True, npairs == 1)
        if npairs > 2:
            def pair_body(i, carry):
                do_pair(i, False, False)
                return carry
            lax.fori_loop(1, npairs - 1, pair_body, 0)
        if npairs > 1:
            do_pair(npairs - 1, False, True)
        gl = 2 * (npairs - 1)
        pl.when(gl < cnt)(lambda: wait_scatters(0))
        pl.when(gl + 1 < cnt)(lambda: wait_scatters(1))

        plsc.subcore_barrier()
        pltpu.sync_copy(acc_sh.at[pl.ds(s * zrows, zrows)],
                        parts_hbm.at[c, pl.ds(s * zrows, zrows)])

    f = pl.kernel(
        body,
        out_type=jax.ShapeDtypeStruct((NC, n_acc, DP), jnp.float32),
        mesh=mesh,
        scratch_types=[
            pltpu.VMEM_SHARED((n_acc, DP), jnp.float32),
            pltpu.VMEM((2, K, SB), jnp.int32),
            pltpu.VMEM((2, K, SB), jnp.int32),
            pltpu.VMEM((2, K, SB, DP), jnp.float32),
            pltpu.SemaphoreType.DMA,
            pltpu.SemaphoreType.DMA,
            pltpu.SemaphoreType.DMA,
            pltpu.SemaphoreType.DMA,
            pltpu.SemaphoreType.DMA,
            pltpu.SemaphoreType.DMA,
        ],
        compiler_params=pltpu.CompilerParams(use_tc_tiling_on_sc=False),
    )
    return f(x_pad, rc, zeros)


L = 16


def _sc_mlp(x_pad, parts, W1, b1, W2, b2, W3, b3, n, n_acc):
    d = 5
    h1 = W1.shape[1]
    h2 = W2.shape[1]
    do = W3.shape[1]
    npt = n // (NC * NS)
    gpt = -(-npt // L)
    npt_pad = gpt * L
    cpt = n // NS
    cchunk = npt
    ncc = cpt // cchunk
    cgroups = -(-cchunk // L)
    mesh = plsc.VectorSubcoreMesh(
        core_axis_name="c", subcore_axis_name="s",
        num_cores=NC, num_subcores=NS)

    def body(x_hbm, parts_hbm, w1_h, b1_h, w2_h, b2_h, w3_h, b3_h, out_hbm,
             stage_sh, xb, p0b, p1b, ob, w1v, b1v, w2v, b2v, w3v, b3v,
             cv, sv, sm):
        c = lax.axis_index("c")
        s = lax.axis_index("s")
        t = c * NS + s
        base = t * npt
        pltpu.async_copy(parts_hbm.at[0, pl.ds(base, npt_pad)], p0b, sm)
        pltpu.async_copy(parts_hbm.at[1, pl.ds(base, npt_pad)], p1b, sm)
        pltpu.sync_copy(w1_h, w1v)
        pltpu.sync_copy(b1_h, b1v)
        pltpu.sync_copy(w2_h, w2v)
        pltpu.sync_copy(b2_h, b2v)
        pltpu.sync_copy(w3_h, w3v)
        pltpu.sync_copy(b3_h, b3v)

        lanes = lax.iota(jnp.int32, L)

        def gat(ref, rows, col):
            return plsc.load_gather(ref, [rows, jnp.full((L,), col,
                                                         jnp.int32)])

        cnt = jnp.zeros((L,), jnp.int32)
        for ch in range(ncc):
            pltpu.sync_copy(x_hbm.at[pl.ds(s * cpt + ch * cchunk, cchunk)],
                            xb.at[pl.ds(0, cchunk)])

            def cgrp(g, acc):
                rows = lanes + g * L
                nz = gat(xb, rows, 0) != 0.0
                for k in range(1, d):
                    nz = nz | (gat(xb, rows, k) != 0.0)
                valid = lanes < (cchunk - g * L)
                return acc + plsc.all_reduce_population_count(nz & valid)

            cnt = lax.fori_loop(0, cgroups, cgrp, cnt)
        cv[...] = cnt
        pltpu.sync_copy(cv, stage_sh.at[s])
        plsc.subcore_barrier()
        pltpu.sync_copy(stage_sh, sv)
        total = sv[0]
        for r in range(1, NS):
            total = total + sv[r]
        inv = 1.0 / jnp.maximum(total.astype(jnp.float32), 1.0)

        pltpu.sync_copy(x_hbm.at[pl.ds(base, npt_pad)], xb)
        pltpu.make_async_copy(parts_hbm.at[0, pl.ds(base, npt_pad)], p0b,
                              sm).wait()
        pltpu.make_async_copy(parts_hbm.at[1, pl.ds(base, npt_pad)], p1b,
                              sm).wait()

        w1s = [[w1v[k][j] for j in range(h1)] for k in range(2 * d)]
        w2s = [[w2v[k][j] for j in range(h2)] for k in range(h1)]
        w3s = [[w3v[k][m] for m in range(do)] for k in range(h2)]
        b1s = [b1v[...][j] for j in range(h1)]
        b2s = [b2v[...][j] for j in range(h2)]
        b3s = [b3v[...][m] for m in range(do)]

        def grp(g2, carry):
          for u in range(2):
            g = 2 * g2 + u
            rows = lanes + g * L
            f = [gat(xb, rows, k) for k in range(d)]
            f += [(gat(p0b, rows, k) + gat(p1b, rows, k)) * inv
                  for k in range(d)]
            hh = []
            for j in range(h1):
                a = b1s[j]
                for k in range(2 * d):
                    a = a + f[k] * w1s[k][j]
                hh.append(jnp.maximum(a, 0.0))
            h2v_ = []
            for j in range(h2):
                a = b2s[j]
                for k in range(h1):
                    a = a + hh[k] * w2s[k][j]
                h2v_.append(jnp.maximum(a, 0.0))
            for m in range(do):
                a = b3s[m]
                for k in range(h2):
                    a = a + h2v_[k] * w3s[k][m]
                plsc.store_scatter(ob, [rows, jnp.full((L,), m, jnp.int32)],
                                   a)
          return carry

        lax.fori_loop(0, gpt // 2, grp, 0)
        pltpu.sync_copy(ob.at[pl.ds(0, npt)], out_hbm.at[pl.ds(base, npt)])

    f = pl.kernel(
        body,
        out_type=jax.ShapeDtypeStruct((n, DP), jnp.float32),
        mesh=mesh,
        scratch_types=[
            pltpu.VMEM_SHARED((NS, L), jnp.int32),
            pltpu.VMEM((npt_pad, DP), jnp.float32),
            pltpu.VMEM((npt_pad, DP), jnp.float32),
            pltpu.VMEM((npt_pad, DP), jnp.float32),
            pltpu.VMEM((npt_pad, DP), jnp.float32),
            pltpu.VMEM((2 * d, h1), jnp.float32),
            pltpu.VMEM((h1,), jnp.float32),
            pltpu.VMEM((h1, h2), jnp.float32),
            pltpu.VMEM((h2,), jnp.float32),
            pltpu.VMEM((h2, L), jnp.float32),
            pltpu.VMEM((L,), jnp.float32),
            pltpu.VMEM((L,), jnp.int32),
            pltpu.VMEM((NS, L), jnp.int32),
            pltpu.SemaphoreType.DMA,
        ],
        compiler_params=pltpu.CompilerParams(use_tc_tiling_on_sc=False,
                                             needs_layout_passes=False),
    )
    w3p = jnp.pad(W3, ((0, 0), (0, L - do)))
    b3p = jnp.pad(b3, (0, L - do))
    return f(x_pad, parts, W1, b1, W2, b2, w3p, b3p)


def kernel(x, edge_index, edge_attr, u, batch, W1, b1, W2, b2, W3, b3):
    n, d = x.shape
    e = edge_index.shape[1]
    ep = K * SB * (-(-e // (K * SB)))
    ei = edge_index if ep == e else jnp.pad(
        edge_index, ((0, 0), (0, ep - e)), constant_values=n)
    rc = ei.reshape(2, ep // SB, SB)
    total_chunks = ep // (K * SB)
    c0_chunks = total_chunks // 2
    npt = n // (NC * NS)
    npt_pad = L * (-(-npt // L))
    nx = max(n, (NC * NS - 1) * npt + npt_pad)
    nx = 8 * (-(-nx // 8))
    x_pad = jnp.pad(x, ((0, nx - n), (0, DP - d)))
    n_acc = 8 * NS * (-(-(n + 1) // (8 * NS)))
    zeros = jnp.zeros((n_acc // NS, DP), jnp.float32)
    parts = _sc_segment_sum(x_pad, rc, zeros, n_acc, total_chunks, c0_chunks)
    out8 = _sc_mlp(x_pad, parts, W1, b1, W2, b2, W3, b3, n, n_acc)
    return out8[:, :W3.shape[1]]


# speedup vs baseline: 1.0149x; 1.0149x over previous
import jax
import jax.numpy as jnp
from jax import lax
from jax.experimental import pallas as pl
from jax.experimental.pallas import tpu as pltpu
from jax.experimental.pallas import tpu_sc as plsc

NC = 2
NS = 16
SB = 128
K = 10
DP = 8


def _sc_segment_sum(x_pad, rc, zeros, n_acc, total_chunks, c0_chunks):
    zrows = n_acc // NS
    mesh = plsc.VectorSubcoreMesh(
        core_axis_name="c", subcore_axis_name="s",
        num_cores=NC, num_subcores=NS)

    q0, r0 = divmod(c0_chunks, NS)
    q1, r1 = divmod(total_chunks - c0_chunks, NS)
    nslot = max(q0 + (1 if r0 else 0), q1 + (1 if r1 else 0))
    npairs = -(-nslot // 2)

    def body(x_hbm, rc_hbm, zeros_hbm, parts_hbm,
             acc_sh, ridx_v, cidx_v, vals_v,
             si0, si1, sg0, sg1, ss0, ss1):
        c = lax.axis_index("c")
        s = lax.axis_index("s")
        si = (si0, si1)
        sg = (sg0, sg1)
        ss = (ss0, ss1)
        cnt = jnp.where(c == 0, q0 + (s < r0).astype(jnp.int32),
                        q1 + (s < r1).astype(jnp.int32))
        start = jnp.where(c == 0, q0 * s + jnp.minimum(s, r0),
                          c0_chunks + q1 * s + jnp.minimum(s, r1))
        pltpu.sync_copy(zeros_hbm, acc_sh.at[pl.ds(s * zrows, zrows)])
        plsc.subcore_barrier()

        def fire_idx(g, b):
            sb = (start + jnp.minimum(g, cnt - 1)) * K
            pltpu.async_copy(rc_hbm.at[0, pl.ds(sb, K)], ridx_v.at[b], si[b])
            pltpu.async_copy(rc_hbm.at[1, pl.ds(sb, K)], cidx_v.at[b], si[b])

        def wait_idx(b):
            pltpu.make_async_copy(rc_hbm.at[0, pl.ds(0, K)], ridx_v.at[b],
                                  si[b]).wait()
            pltpu.make_async_copy(rc_hbm.at[1, pl.ds(0, K)], cidx_v.at[b],
                                  si[b]).wait()

        def fire_gathers(b):
            for j in range(K):
                pltpu.async_copy(x_hbm.at[ridx_v.at[b, j]], vals_v.at[b, j],
                                 sg[b])

        def wait_gathers(b):
            for j in range(K):
                pltpu.make_async_copy(x_hbm.at[ridx_v.at[b, j]],
                                      vals_v.at[b, j], sg[b]).wait()

        def fire_scatters(b):
            for j in range(K):
                pltpu.async_copy(vals_v.at[b, j], acc_sh.at[cidx_v.at[b, j]],
                                 ss[b], add=True)

        def wait_scatters(b):
            for j in range(K):
                pltpu.make_async_copy(vals_v.at[b, j],
                                      acc_sh.at[cidx_v.at[b, j]],
                                      ss[b]).wait()

        def do_pair(i, first, last):
            g0 = 2 * i
            g1 = g0 + 1
            wait_idx(0)
            if not first:
                pl.when(g0 - 2 < cnt)(lambda: wait_scatters(0))
            pl.when(g0 < cnt)(lambda: fire_gathers(0))
            wait_idx(1)
            if not first:
                pl.when(g1 - 2 < cnt)(lambda: wait_scatters(1))

            def _drain0():
                wait_gathers(0)
                fire_scatters(0)
            pl.when(g0 < cnt)(_drain0)
            pl.when(g1 < cnt)(lambda: fire_gathers(1))
            if not last:
                fire_idx(g0 + 2, 0)

            def _drain1():
                wait_gathers(1)
                fire_scatters(1)
            pl.when(g1 < cnt)(_drain1)
            if not last:
                fire_idx(g1 + 2, 1)

        fire_idx(0, 0)
        fire_idx(1, 1)
        do_pair(0, True, npairs == 1)
        if npairs > 2:
            def pair_body(i, carry):
                do_pair(i, False, False)
                return carry
            lax.fori_loop(1, npairs - 1, pair_body, 0)
        if npairs > 1:
            do_pair(npairs - 1, False, True)
        gl = 2 * (npairs - 1)
        pl.when(gl < cnt)(lambda: wait_scatters(0))
        pl.when(gl + 1 < cnt)(lambda: wait_scatters(1))

        plsc.subcore_barrier()
        pltpu.sync_copy(acc_sh.at[pl.ds(s * zrows, zrows)],
                        parts_hbm.at[c, pl.ds(s * zrows, zrows)])

    f = pl.kernel(
        body,
        out_type=jax.ShapeDtypeStruct((NC, n_acc, DP), jnp.float32),
        mesh=mesh,
        scratch_types=[
            pltpu.VMEM_SHARED((n_acc, DP), jnp.float32),
            pltpu.VMEM((2, K, SB), jnp.int32),
            pltpu.VMEM((2, K, SB), jnp.int32),
            pltpu.VMEM((2, K, SB, DP), jnp.float32),
            pltpu.SemaphoreType.DMA,
            pltpu.SemaphoreType.DMA,
            pltpu.SemaphoreType.DMA,
            pltpu.SemaphoreType.DMA,
            pltpu.SemaphoreType.DMA,
            pltpu.SemaphoreType.DMA,
        ],
        compiler_params=pltpu.CompilerParams(use_tc_tiling_on_sc=False),
    )
    return f(x_pad, rc, zeros)


L = 16


def _sc_mlp(x_pad, parts, W1, b1, W2, b2, W3, b3, n, n_acc):
    d = 5
    h1 = W1.shape[1]
    h2 = W2.shape[1]
    do = W3.shape[1]
    npt = n // (NC * NS)
    gpt = -(-npt // L)
    npt_pad = gpt * L
    cpt = n // NS
    cchunk = npt
    ncc = cpt // cchunk
    cgroups = -(-cchunk // L)
    mesh = plsc.VectorSubcoreMesh(
        core_axis_name="c", subcore_axis_name="s",
        num_cores=NC, num_subcores=NS)

    def body(x_hbm, parts_hbm, w1_h, b1_h, w2_h, b2_h, w3_h, b3_h, out_hbm,
             stage_sh, xb, p0b, p1b, ob, w1v, b1v, w2v, b2v, w3v, b3v,
             cv, sv, sm):
        c = lax.axis_index("c")
        s = lax.axis_index("s")
        t = c * NS + s
        base = t * npt
        pltpu.async_copy(parts_hbm.at[0, pl.ds(base, npt_pad)], p0b, sm)
        pltpu.async_copy(parts_hbm.at[1, pl.ds(base, npt_pad)], p1b, sm)
        pltpu.sync_copy(w1_h, w1v)
        pltpu.sync_copy(b1_h, b1v)
        pltpu.sync_copy(w2_h, w2v)
        pltpu.sync_copy(b2_h, b2v)
        pltpu.sync_copy(w3_h, w3v)
        pltpu.sync_copy(b3_h, b3v)

        lanes = lax.iota(jnp.int32, L)

        def gat(ref, rows, col):
            return plsc.load_gather(ref, [rows, jnp.full((L,), col,
                                                         jnp.int32)])

        cnt = jnp.zeros((L,), jnp.int32)
        for ch in range(ncc):
            pltpu.sync_copy(x_hbm.at[pl.ds(s * cpt + ch * cchunk, cchunk)],
                            xb.at[pl.ds(0, cchunk)])

            def cgrp(g, acc):
                rows = lanes + g * L
                nz = gat(xb, rows, 0) != 0.0
                for k in range(1, d):
                    nz = nz | (gat(xb, rows, k) != 0.0)
                valid = lanes < (cchunk - g * L)
                return acc + plsc.all_reduce_population_count(nz & valid)

            cnt = lax.fori_loop(0, cgroups, cgrp, cnt)
        cv[...] = cnt
        pltpu.sync_copy(cv, stage_sh.at[s])
        plsc.subcore_barrier()
        pltpu.sync_copy(stage_sh, sv)
        total = sv[0]
        for r in range(1, NS):
            total = total + sv[r]
        inv = 1.0 / jnp.maximum(total.astype(jnp.float32), 1.0)

        pltpu.sync_copy(x_hbm.at[pl.ds(base, npt_pad)], xb)
        pltpu.make_async_copy(parts_hbm.at[0, pl.ds(base, npt_pad)], p0b,
                              sm).wait()
        pltpu.make_async_copy(parts_hbm.at[1, pl.ds(base, npt_pad)], p1b,
                              sm).wait()

        w1s = [[w1v[k][j] for j in range(h1)] for k in range(2 * d)]
        w2s = [[w2v[k][j] for j in range(h2)] for k in range(h1)]
        w3s = [[w3v[k][m] for m in range(do)] for k in range(h2)]
        b1s = [b1v[...][j] for j in range(h1)]
        b2s = [b2v[...][j] for j in range(h2)]
        b3s = [b3v[...][m] for m in range(do)]

        def grp(g2, carry):
          for u in range(2):
            g = 2 * g2 + u
            rows = lanes + g * L
            f = [gat(xb, rows, k) for k in range(d)]
            f += [(gat(p0b, rows, k) + gat(p1b, rows, k)) * inv
                  for k in range(d)]
            hh = []
            for j in range(h1):
                a = b1s[j]
                for k in range(2 * d):
                    a = a + f[k] * w1s[k][j]
                hh.append(jnp.maximum(a, 0.0))
            h2v_ = []
            for j in range(h2):
                a = b2s[j]
                for k in range(h1):
                    a = a + hh[k] * w2s[k][j]
                h2v_.append(jnp.maximum(a, 0.0))
            for m in range(do):
                a = b3s[m]
                for k in range(h2):
                    a = a + h2v_[k] * w3s[k][m]
                plsc.store_scatter(ob, [rows, jnp.full((L,), m, jnp.int32)],
                                   a)
          return carry

        lax.fori_loop(0, gpt // 2, grp, 0)
        pltpu.sync_copy(ob.at[pl.ds(0, npt)], out_hbm.at[pl.ds(base, npt)])

    f = pl.kernel(
        body,
        out_type=jax.ShapeDtypeStruct((n, DP), jnp.float32),
        mesh=mesh,
        scratch_types=[
            pltpu.VMEM_SHARED((NS, L), jnp.int32),
            pltpu.VMEM((npt_pad, DP), jnp.float32),
            pltpu.VMEM((npt_pad, DP), jnp.float32),
            pltpu.VMEM((npt_pad, DP), jnp.float32),
            pltpu.VMEM((npt_pad, DP), jnp.float32),
            pltpu.VMEM((2 * d, h1), jnp.float32),
            pltpu.VMEM((h1,), jnp.float32),
            pltpu.VMEM((h1, h2), jnp.float32),
            pltpu.VMEM((h2,), jnp.float32),
            pltpu.VMEM((h2, L), jnp.float32),
            pltpu.VMEM((L,), jnp.float32),
            pltpu.VMEM((L,), jnp.int32),
            pltpu.VMEM((NS, L), jnp.int32),
            pltpu.SemaphoreType.DMA,
        ],
        compiler_params=pltpu.CompilerParams(use_tc_tiling_on_sc=False,
                                             needs_layout_passes=False),
    )
    w3p = jnp.pad(W3, ((0, 0), (0, L - do)))
    b3p = jnp.pad(b3, (0, L - do))
    return f(x_pad, parts, W1, b1, W2, b2, w3p, b3p)


def kernel(x, edge_index, edge_attr, u, batch, W1, b1, W2, b2, W3, b3):
    n, d = x.shape
    e = edge_index.shape[1]
    ep = K * SB * (-(-e // (K * SB)))
    ei = edge_index if ep == e else jnp.pad(
        edge_index, ((0, 0), (0, ep - e)), constant_values=n)
    rc = ei.reshape(2, ep // SB, SB)
    total_chunks = ep // (K * SB)
    c0_chunks = total_chunks // 2
    npt = n // (NC * NS)
    npt_pad = L * (-(-npt // L))
    nx = max(n, (NC * NS - 1) * npt + npt_pad)
    nx = 8 * (-(-nx // 8))
    x_pad = jnp.pad(x, ((0, nx - n), (0, DP - d)))
    n_acc = 8 * NS * (-(-(n + 1) // (8 * NS)))
    zeros = jnp.zeros((n_acc // NS, DP), jnp.float32)
    parts = _sc_segment_sum(x_pad, rc, zeros, n_acc, total_chunks, c0_chunks)
    out8 = _sc_mlp(x_pad, parts, W1, b1, W2, b2, W3, b3, n, n_acc)
    return out8[:, :W3.shape[1]]
